# Initial kernel scaffold; baseline (speedup 1.0000x reference)
import jax
import jax.numpy as jnp
from jax.experimental import pallas as pl


def kernel(x, edge_index, edge_attr, batch, node_table, edge_table, W1_p, b1_p, g1_p, be1_p, W2_p, b2_p, g2_p, be2_p, W1_c, b1_c, g1_c, be1_c, W2_c, b2_c, g2_c, be2_c, Wf, bf, gf, bef):
    raise NotImplementedError("write your pallas kernel here")


# speedup vs baseline: 1.9755x; 1.9755x over previous
import jax
import jax.numpy as jnp
from jax import lax
from jax.experimental import pallas as pl
from jax.experimental.pallas import tpu as pltpu
from jax.experimental.pallas import tpu_sc as plsc

N = 10000
E = 320000
D = 128
VOCAB = 2000
MAX_EDGES = 200
NUM_GRAPHS = 16
NUM_ITER = 2
EPS = 1e-5

NC = 2
NS = 16
NW = NC * NS

CB_N = 80
NCH_N = N // CB_N
CB_E = 80
EPT = E // NS
NCH_DEG = EPT // CB_E

CB_B = 80
EPW = E // NW
NCH_B = EPW // CB_B

CB_D = 80
NCH_D = EPT // CB_D

BE = 2000
NBLK = E // BE

_mesh = plsc.VectorSubcoreMesh(core_axis_name="c", subcore_axis_name="s")


def _sc_prologue(x_hbm, nt_hbm, dst_hbm, src_hbm,
                 nodes_hbm, degd_hbm, degs_hbm,
                 xidx, rows, eidx, ones, acc, sem):
    c = lax.axis_index("c")
    s = lax.axis_index("s")
    wid = s * NC + c

    def fill(j, _):
        for k in range(8):
            ones[j, pl.ds(k * 16, 16)] = jnp.ones((16,), jnp.float32)
            rows[j, pl.ds(k * 16, 16)] = jnp.zeros((16,), jnp.float32)
        return 0
    lax.fori_loop(0, CB_N, fill, 0)

    for i in range(8):
        ch = s + i * NS

        @pl.when(ch < NCH_N)
        def _():
            pltpu.sync_copy(rows, acc.at[pl.ds(ch * CB_N, CB_N)])

    plsc.subcore_barrier()

    def node_chunk(i, _):
        st = (wid + i * NW) * CB_N
        pltpu.sync_copy(x_hbm.at[pl.ds(st, CB_N)], xidx)
        pltpu.async_copy(nt_hbm.at[xidx], rows, sem).wait()
        pltpu.sync_copy(rows, nodes_hbm.at[pl.ds(st, CB_N)])
        return 0
    lax.fori_loop(0, jnp.where(wid < NCH_N - 3 * NW, 4, 3), node_chunk, 0)

    def deg_loop(idx_hbm):
        def chunk(i, _):
            st = s * EPT + i * CB_E
            pltpu.sync_copy(idx_hbm.at[pl.ds(st, CB_E)], eidx)
            pltpu.sync_copy(ones, acc.at[eidx], add=True)
            return 0
        lax.fori_loop(0, NCH_DEG, chunk, 0)

    @pl.when(c == 0)
    def _():
        deg_loop(dst_hbm)

    @pl.when(c == 1)
    def _():
        deg_loop(src_hbm)

    plsc.subcore_barrier()

    def deg_out(out_hbm):
        for i in range(8):
            ch = s + i * NS

            @pl.when(ch < NCH_N)
            def _():
                pltpu.sync_copy(acc.at[pl.ds(ch * CB_N, CB_N)], rows)
                pltpu.sync_copy(rows, out_hbm.at[pl.ds(ch * CB_N, CB_N)])

    @pl.when(c == 0)
    def _():
        deg_out(degd_hbm)

    @pl.when(c == 1)
    def _():
        deg_out(degs_hbm)


_prologue_call = pl.kernel(
    _sc_prologue,
    out_type=[
        jax.ShapeDtypeStruct((N, D), jnp.float32),
        jax.ShapeDtypeStruct((N, D), jnp.float32),
        jax.ShapeDtypeStruct((N, D), jnp.float32),
    ],
    mesh=_mesh,
    scratch_types=[
        pltpu.VMEM((CB_N,), jnp.int32),
        pltpu.VMEM((CB_N, D), jnp.float32),
        pltpu.VMEM((CB_E,), jnp.int32),
        pltpu.VMEM((CB_E, D), jnp.float32),
        pltpu.VMEM_SHARED((N, D), jnp.float32),
        pltpu.SemaphoreType.DMA,
    ],
    name="sc_prologue",
)


def _sc_gather_sum(pdp_hbm, pdc_hbm, psp_hbm, psc_hbm, pap_hbm, pac_hbm,
                   dst_hbm, src_hbm, attr_hbm,
                   h1p_hbm, h1c_hbm, stats_hbm,
                   idxD, idxS, idxA, bDP, bDC, bSP, bSC, bAP, bAC,
                   outP, outC, stage, semD, semS, semA):
    c = lax.axis_index("c")
    s = lax.axis_index("s")
    wid = s * NC + c
    base = wid * EPW

    zero = jnp.zeros((16,), jnp.float32)
    init = tuple(zero for _ in range(32))

    def chunk_body(i, acc):
        st = base + i * CB_B
        pltpu.sync_copy(dst_hbm.at[pl.ds(st, CB_B)], idxD)
        pltpu.sync_copy(src_hbm.at[pl.ds(st, CB_B)], idxS)
        pltpu.sync_copy(attr_hbm.at[pl.ds(st, CB_B)], idxA)
        cps = [pltpu.async_copy(pdp_hbm.at[idxD], bDP, semD),
               pltpu.async_copy(pdc_hbm.at[idxD], bDC, semD),
               pltpu.async_copy(psp_hbm.at[idxS], bSP, semS),
               pltpu.async_copy(psc_hbm.at[idxS], bSC, semS),
               pltpu.async_copy(pap_hbm.at[idxA], bAP, semA),
               pltpu.async_copy(pac_hbm.at[idxA], bAC, semA)]
        for cp in cps:
            cp.wait()

        def row_body(j, a):
            a = list(a)
            for k in range(8):
                sl = pl.ds(k * 16, 16)
                v = bDP[j, sl] + bSP[j, sl] + bAP[j, sl]
                outP[j, sl] = v
                a[k] = a[k] + v
                a[8 + k] = a[8 + k] + v * v
                w = bDC[j, sl] + bSC[j, sl] + bAC[j, sl]
                outC[j, sl] = w
                a[16 + k] = a[16 + k] + w
                a[24 + k] = a[24 + k] + w * w
            return tuple(a)

        acc = lax.fori_loop(0, CB_B, row_body, acc)
        pltpu.sync_copy(outP, h1p_hbm.at[pl.ds(st, CB_B)])
        pltpu.sync_copy(outC, h1c_hbm.at[pl.ds(st, CB_B)])
        return acc

    acc = lax.fori_loop(0, NCH_B, chunk_body, init)
    for k in range(8):
        sl = pl.ds(k * 16, 16)
        stage[0, sl] = acc[k]
        stage[1, sl] = acc[8 + k]
        stage[2, sl] = acc[16 + k]
        stage[3, sl] = acc[24 + k]
        for r in range(4, 8):
            stage[r, sl] = zero
    pltpu.sync_copy(stage, stats_hbm.at[wid])


_gather_sum_call = pl.kernel(
    _sc_gather_sum,
    out_type=[
        jax.ShapeDtypeStruct((E, D), jnp.float32),
        jax.ShapeDtypeStruct((E, D), jnp.float32),
        jax.ShapeDtypeStruct((NW, 8, D), jnp.float32),
    ],
    mesh=_mesh,
    scratch_types=[
        pltpu.VMEM((CB_B,), jnp.int32),
        pltpu.VMEM((CB_B,), jnp.int32),
        pltpu.VMEM((CB_B,), jnp.int32),
        pltpu.VMEM((CB_B, D), jnp.float32),
        pltpu.VMEM((CB_B, D), jnp.float32),
        pltpu.VMEM((CB_B, D), jnp.float32),
        pltpu.VMEM((CB_B, D), jnp.float32),
        pltpu.VMEM((CB_B, D), jnp.float32),
        pltpu.VMEM((CB_B, D), jnp.float32),
        pltpu.VMEM((CB_B, D), jnp.float32),
        pltpu.VMEM((CB_B, D), jnp.float32),
        pltpu.VMEM((8, D), jnp.float32),
        pltpu.SemaphoreType.DMA,
        pltpu.SemaphoreType.DMA,
        pltpu.SemaphoreType.DMA,
    ],
    name="sc_gather_sum",
)


def _sc_scatter(h2p_hbm, h2c_hbm, coef_hbm, dst_hbm, src_hbm,
                aggp_hbm, aggc_hbm,
                buf, idx, coefv, acc, sem):
    c = lax.axis_index("c")
    s = lax.axis_index("s")

    def zero_buf(j, _):
        for k in range(8):
            buf[j, pl.ds(k * 16, 16)] = jnp.zeros((16,), jnp.float32)
        return 0
    lax.fori_loop(0, CB_D, zero_buf, 0)

    for i in range(8):
        ch = s + i * NS

        @pl.when(ch < NCH_N)
        def _():
            pltpu.sync_copy(buf, acc.at[pl.ds(ch * CB_N, CB_N)])

    plsc.subcore_barrier()

    pltpu.sync_copy(coef_hbm, coefv)

    def scatter_loop(h2_hbm, idx_hbm, arow, crow):
        av = [coefv[arow, pl.ds(k * 16, 16)] for k in range(8)]
        cv = [coefv[crow, pl.ds(k * 16, 16)] for k in range(8)]
        ebase = s * EPT

        def chunk(i, _):
            st = ebase + i * CB_D
            pltpu.sync_copy(idx_hbm.at[pl.ds(st, CB_D)], idx)
            pltpu.sync_copy(h2_hbm.at[pl.ds(st, CB_D)], buf)

            def row(j, __):
                for k in range(8):
                    sl = pl.ds(k * 16, 16)
                    buf[j, sl] = jnp.maximum(buf[j, sl] * av[k] + cv[k], 0.0)
                return 0
            lax.fori_loop(0, CB_D, row, 0)
            pltpu.sync_copy(buf, acc.at[idx], add=True)
            return 0
        lax.fori_loop(0, NCH_D, chunk, 0)

    @pl.when(c == 0)
    def _():
        scatter_loop(h2p_hbm, dst_hbm, 0, 1)

    @pl.when(c == 1)
    def _():
        scatter_loop(h2c_hbm, src_hbm, 2, 3)

    plsc.subcore_barrier()

    def agg_out(out_hbm):
        for i in range(8):
            ch = s + i * NS

            @pl.when(ch < NCH_N)
            def _():
                pltpu.sync_copy(acc.at[pl.ds(ch * CB_N, CB_N)], buf)
                pltpu.sync_copy(buf, out_hbm.at[pl.ds(ch * CB_N, CB_N)])

    @pl.when(c == 0)
    def _():
        agg_out(aggp_hbm)

    @pl.when(c == 1)
    def _():
        agg_out(aggc_hbm)


_scatter_call = pl.kernel(
    _sc_scatter,
    out_type=[
        jax.ShapeDtypeStruct((N, D), jnp.float32),
        jax.ShapeDtypeStruct((N, D), jnp.float32),
    ],
    mesh=_mesh,
    scratch_types=[
        pltpu.VMEM((CB_D, D), jnp.float32),
        pltpu.VMEM((CB_D,), jnp.int32),
        pltpu.VMEM((4, D), jnp.float32),
        pltpu.VMEM_SHARED((N, D), jnp.float32),
        pltpu.SemaphoreType.DMA,
    ],
    name="sc_scatter",
)


def _tc_proj(nodes, wdp, wdc, wsp, wsc, et, wap, wac, pdp, pdc, psp, psc,
             pap, pac):
    n = nodes[...]
    pdp[...] = jnp.dot(n, wdp[...], preferred_element_type=jnp.float32)
    pdc[...] = jnp.dot(n, wdc[...], preferred_element_type=jnp.float32)
    psp[...] = jnp.dot(n, wsp[...], preferred_element_type=jnp.float32)
    psc[...] = jnp.dot(n, wsc[...], preferred_element_type=jnp.float32)
    e = et[...]
    pap[...] = jnp.dot(e, wap[...], preferred_element_type=jnp.float32)
    pac[...] = jnp.dot(e, wac[...], preferred_element_type=jnp.float32)


def _run_tc_proj(nodes, wdp, wdc, wsp, wsc, et, wap, wac):
    return pl.pallas_call(
        _tc_proj,
        out_shape=[
            jax.ShapeDtypeStruct((N, D), jnp.float32),
            jax.ShapeDtypeStruct((N, D), jnp.float32),
            jax.ShapeDtypeStruct((N, D), jnp.float32),
            jax.ShapeDtypeStruct((N, D), jnp.float32),
            jax.ShapeDtypeStruct((MAX_EDGES, D), jnp.float32),
            jax.ShapeDtypeStruct((MAX_EDGES, D), jnp.float32),
        ],
        name="tc_proj",
    )(nodes, wdp, wdc, wsp, wsc, et, wap, wac)


def _tc_mlp2(h1p, h1c, stats1, w2p, b2p, g2p, be2p, g1p, be1p,
             w2c, b2c, g2c, be2c, g1c, be1c,
             h2p, h2c, stats2, coef):
    i = pl.program_id(0)
    s1 = jnp.sum(stats1[...], axis=0)
    fE = float(E)

    def bn_coef(su, sq, g, be):
        mu = su / fE
        var = sq / fE - mu * mu
        a = g * lax.rsqrt(var + EPS)
        return a, be - a * mu

    a1p, c1p = bn_coef(s1[0:1, :], s1[1:2, :], g1p[...], be1p[...])
    a1c, c1c = bn_coef(s1[2:3, :], s1[3:4, :], g1c[...], be1c[...])

    zp = jnp.maximum(a1p * h1p[...] + c1p, 0.0)
    zc = jnp.maximum(a1c * h1c[...] + c1c, 0.0)
    vp = jnp.dot(zp, w2p[...], preferred_element_type=jnp.float32) + b2p[...]
    vc = jnp.dot(zc, w2c[...], preferred_element_type=jnp.float32) + b2c[...]
    h2p[...] = vp
    h2c[...] = vc

    bs = jnp.concatenate(
        [jnp.concatenate([jnp.sum(vp, 0, keepdims=True),
                          jnp.sum(vc, 0, keepdims=True)], axis=1),
         jnp.concatenate([jnp.sum(vp * vp, 0, keepdims=True),
                          jnp.sum(vc * vc, 0, keepdims=True)], axis=1)],
        axis=0)

    @pl.when(i == 0)
    def _():
        stats2[...] = bs

    @pl.when(i > 0)
    def _():
        stats2[...] += bs

    tot = stats2[...]
    a2p, c2p = bn_coef(tot[0:1, :D], tot[1:2, :D], g2p[...], be2p[...])
    a2c, c2c = bn_coef(tot[0:1, D:], tot[1:2, D:], g2c[...], be2c[...])
    coef[...] = jnp.concatenate([a2p, c2p, a2c, c2c], axis=0)


def _run_tc_mlp2(h1p, h1c, stats1, w2p, b2p, g2p, be2p, g1p, be1p,
                 w2c, b2c, g2c, be2c, g1c, be1c):
    blk = pl.BlockSpec((BE, D), lambda i: (i, 0))
    whole2 = pl.BlockSpec((1, D), lambda i: (0, 0))
    wmat = pl.BlockSpec((D, D), lambda i: (0, 0))
    return pl.pallas_call(
        _tc_mlp2,
        grid=(NBLK,),
        in_specs=[blk, blk, pl.BlockSpec((NW, 8, D), lambda i: (0, 0, 0)),
                  wmat, whole2, whole2, whole2, whole2, whole2,
                  wmat, whole2, whole2, whole2, whole2, whole2],
        out_specs=[blk, blk,
                   pl.BlockSpec((2, 2 * D), lambda i: (0, 0)),
                   pl.BlockSpec((4, D), lambda i: (0, 0))],
        out_shape=[
            jax.ShapeDtypeStruct((E, D), jnp.float32),
            jax.ShapeDtypeStruct((E, D), jnp.float32),
            jax.ShapeDtypeStruct((2, 2 * D), jnp.float32),
            jax.ShapeDtypeStruct((4, D), jnp.float32),
        ],
        name="tc_mlp2",
    )(h1p, h1c, stats1, w2p, b2p, g2p, be2p, g1p, be1p,
      w2c, b2c, g2c, be2c, g1c, be1c)


BN = 1000
NBN = N // BN


def _tc_update1(nodes, aggp, aggc, degp, degc, wf, bf, h_out, stats):
    i = pl.program_id(0)
    sump = jnp.sum(degp[...], axis=1, keepdims=True)
    sumc = jnp.sum(degc[...], axis=1, keepdims=True)
    invp = jnp.where(sump > 0, float(D) / sump, 0.0)
    invc = jnp.where(sumc > 0, float(D) / sumc, 0.0)
    s = nodes[...] + aggp[...] * invp + aggc[...] * invc
    h = jnp.dot(s, wf[...], preferred_element_type=jnp.float32) + bf[...]
    h_out[...] = h
    bs = jnp.concatenate([jnp.sum(h, 0, keepdims=True),
                          jnp.sum(h * h, 0, keepdims=True)], axis=0)

    @pl.when(i == 0)
    def _():
        stats[...] = bs

    @pl.when(i > 0)
    def _():
        stats[...] += bs


def _tc_update2(nodes, h, stats, gf, bef, nodes_new):
    fN = float(N)
    tot = stats[...]
    mu = tot[0:1, :] / fN
    var = tot[1:2, :] / fN - mu * mu
    a = gf[...] * lax.rsqrt(var + EPS)
    c = bef[...] - a * mu
    upd = jnp.maximum(a * h[...] + c, 0.0)
    nodes_new[...] = nodes[...] + upd


def _run_tc_update(nodes, aggp, aggc, degp, degc, wf, bf, gf, bef):
    blk = pl.BlockSpec((BN, D), lambda i: (i, 0))
    whole2 = pl.BlockSpec((1, D), lambda i: (0, 0))
    h, stats = pl.pallas_call(
        _tc_update1,
        grid=(NBN,),
        in_specs=[blk, blk, blk, blk, blk,
                  pl.BlockSpec((D, D), lambda i: (0, 0)), whole2],
        out_specs=[blk, pl.BlockSpec((2, D), lambda i: (0, 0))],
        out_shape=[
            jax.ShapeDtypeStruct((N, D), jnp.float32),
            jax.ShapeDtypeStruct((2, D), jnp.float32),
        ],
        name="tc_update1",
    )(nodes, aggp, aggc, degp, degc, wf, bf)
    return pl.pallas_call(
        _tc_update2,
        grid=(NBN,),
        in_specs=[blk, blk, pl.BlockSpec((2, D), lambda i: (0, 0)),
                  whole2, whole2],
        out_specs=blk,
        out_shape=jax.ShapeDtypeStruct((N, D), jnp.float32),
        name="tc_update2",
    )(nodes, h, stats, gf, bef)


def _tc_segmax(nodes, batch2d, gmax):
    g = pl.program_id(0)
    neg = jnp.float32(-jnp.inf)
    m = jnp.where(batch2d[...] == g, nodes[...], neg)
    gmax[...] = jnp.max(m, axis=0, keepdims=True)[None]


def _run_tc_segmax(nodes, batch2d):
    out = pl.pallas_call(
        _tc_segmax,
        grid=(NUM_GRAPHS,),
        in_specs=[pl.BlockSpec((N, D), lambda g: (0, 0)),
                  pl.BlockSpec((N, 1), lambda g: (0, 0))],
        out_specs=pl.BlockSpec((1, 1, D), lambda g: (g, 0, 0)),
        out_shape=jax.ShapeDtypeStruct((NUM_GRAPHS, 1, D), jnp.float32),
        name="tc_segmax",
    )(nodes, batch2d)
    return out.reshape(NUM_GRAPHS, D)


def kernel(x, edge_index, edge_attr, batch, node_table, edge_table,
           W1_p, b1_p, g1_p, be1_p, W2_p, b2_p, g2_p, be2_p,
           W1_c, b1_c, g1_c, be1_c, W2_c, b2_c, g2_c, be2_c,
           Wf, bf, gf, bef):
    dst = edge_index[1]
    src = edge_index[0]
    attr = edge_attr.astype(jnp.int32)
    xi = x.astype(jnp.int32)
    batch2d = batch.astype(jnp.int32)[:, None]

    wdp = W1_p[0:D, :]
    wdc = W1_c[D:2 * D, :]
    wsp = W1_p[D:2 * D, :]
    wsc = W1_c[0:D, :]
    wap = W1_p[2 * D:, :]
    wac = W1_c[2 * D:, :]

    r2 = lambda v: v[None, :]

    nodes, degp, degc = _prologue_call(xi, node_table, dst, src)

    gmax = None
    for _ in range(NUM_ITER):
        pdp, pdc, psp, psc, pap, pac = _run_tc_proj(
            nodes, wdp, wdc, wsp, wsc, edge_table, wap, wac)
        pap = pap + r2(b1_p)
        pac = pac + r2(b1_c)
        h1p, h1c, stats1 = _gather_sum_call(
            pdp, pdc, psp, psc, pap, pac, dst, src, attr)
        h2p, h2c, _, coef = _run_tc_mlp2(
            h1p, h1c, stats1,
            W2_p, r2(b2_p), r2(g2_p), r2(be2_p), r2(g1_p), r2(be1_p),
            W2_c, r2(b2_c), r2(g2_c), r2(be2_c), r2(g1_c), r2(be1_c))
        aggp, aggc = _scatter_call(h2p, h2c, coef, dst, src)
        nodes = _run_tc_update(nodes, aggp, aggc, degp, degc,
                               Wf, r2(bf), r2(gf), r2(bef))
    return _run_tc_segmax(nodes, batch2d)


# speedup vs baseline: 2.9035x; 1.4697x over previous
import jax
import jax.numpy as jnp
from jax import lax
from jax.experimental import pallas as pl
from jax.experimental.pallas import tpu as pltpu
from jax.experimental.pallas import tpu_sc as plsc

N = 10000
E = 320000
D = 128
VOCAB = 2000
MAX_EDGES = 200
NUM_GRAPHS = 16
NUM_ITER = 2
EPS = 1e-5

NC = 2
NS = 16
NW = NC * NS

CB_N = 80
NCH_N = N // CB_N
CB_E = 80
EPT = E // NS
NCH_DEG = EPT // CB_E

CB_B = 40
EPW = E // NW
NCH_B = EPW // CB_B

CB_D = 80
NCH_D = EPT // CB_D

BE = 2000
NBLK = E // BE

_mesh = plsc.VectorSubcoreMesh(core_axis_name="c", subcore_axis_name="s")


def _sc_prologue(x_hbm, nt_hbm, dst_hbm, src_hbm,
                 nodes_hbm, degd_hbm, degs_hbm,
                 xidx, rows, eidx, ones, acc, sem):
    c = lax.axis_index("c")
    s = lax.axis_index("s")
    wid = s * NC + c

    def fill(j, _):
        for k in range(8):
            ones[j, pl.ds(k * 16, 16)] = jnp.ones((16,), jnp.float32)
            rows[j, pl.ds(k * 16, 16)] = jnp.zeros((16,), jnp.float32)
        return 0
    lax.fori_loop(0, CB_N, fill, 0)

    for i in range(8):
        ch = s + i * NS

        @pl.when(ch < NCH_N)
        def _():
            pltpu.sync_copy(rows, acc.at[pl.ds(ch * CB_N, CB_N)])

    plsc.subcore_barrier()

    def node_chunk(i, _):
        st = (wid + i * NW) * CB_N
        pltpu.sync_copy(x_hbm.at[pl.ds(st, CB_N)], xidx)
        pltpu.async_copy(nt_hbm.at[xidx], rows, sem).wait()
        pltpu.sync_copy(rows, nodes_hbm.at[pl.ds(st, CB_N)])
        return 0
    lax.fori_loop(0, jnp.where(wid < NCH_N - 3 * NW, 4, 3), node_chunk, 0)

    def deg_loop(idx_hbm):
        def chunk(i, _):
            st = s * EPT + i * CB_E
            pltpu.sync_copy(idx_hbm.at[pl.ds(st, CB_E)], eidx)
            pltpu.sync_copy(ones, acc.at[eidx], add=True)
            return 0
        lax.fori_loop(0, NCH_DEG, chunk, 0)

    @pl.when(c == 0)
    def _():
        deg_loop(dst_hbm)

    @pl.when(c == 1)
    def _():
        deg_loop(src_hbm)

    plsc.subcore_barrier()

    def deg_out(out_hbm):
        for i in range(8):
            ch = s + i * NS

            @pl.when(ch < NCH_N)
            def _():
                pltpu.sync_copy(acc.at[pl.ds(ch * CB_N, CB_N)], rows)
                pltpu.sync_copy(rows, out_hbm.at[pl.ds(ch * CB_N, CB_N)])

    @pl.when(c == 0)
    def _():
        deg_out(degd_hbm)

    @pl.when(c == 1)
    def _():
        deg_out(degs_hbm)


_prologue_call = pl.kernel(
    _sc_prologue,
    out_type=[
        jax.ShapeDtypeStruct((N, D), jnp.float32),
        jax.ShapeDtypeStruct((N, D), jnp.float32),
        jax.ShapeDtypeStruct((N, D), jnp.float32),
    ],
    mesh=_mesh,
    scratch_types=[
        pltpu.VMEM((CB_N,), jnp.int32),
        pltpu.VMEM((CB_N, D), jnp.float32),
        pltpu.VMEM((CB_E,), jnp.int32),
        pltpu.VMEM((CB_E, D), jnp.float32),
        pltpu.VMEM_SHARED((N, D), jnp.float32),
        pltpu.SemaphoreType.DMA,
    ],
    name="sc_prologue",
)


def _sc_gather_sum(pdp_hbm, pdc_hbm, psp_hbm, psc_hbm, pap_hbm, pac_hbm,
                   dst_hbm, src_hbm, attr_hbm,
                   h1p_hbm, h1c_hbm, stats_hbm,
                   idxDf, idxSf, idxAf,
                   bDP0, bDC0, bSP0, bSC0, bAP0, bAC0,
                   bDP1, bDC1, bSP1, bSC1, bAP1, bAC1,
                   stage, semG0, semG1, semW0, semW1):
    c = lax.axis_index("c")
    s = lax.axis_index("s")
    wid = s * NC + c
    base = wid * EPW

    bufs = [(bDP0, bDC0, bSP0, bSC0, bAP0, bAC0),
            (bDP1, bDC1, bSP1, bSC1, bAP1, bAC1)]
    semsG = [semG0, semG1]
    semsW = [semW0, semW1]

    pltpu.sync_copy(dst_hbm.at[pl.ds(base, EPW)], idxDf)
    pltpu.sync_copy(src_hbm.at[pl.ds(base, EPW)], idxSf)
    pltpu.sync_copy(attr_hbm.at[pl.ds(base, EPW)], idxAf)

    def issue_gathers(j, b):
        sl = pl.ds(j * CB_B, CB_B)
        B = bufs[b]
        sem = semsG[b]
        pltpu.async_copy(pdp_hbm.at[idxDf.at[sl]], B[0], sem)
        pltpu.async_copy(pdc_hbm.at[idxDf.at[sl]], B[1], sem)
        pltpu.async_copy(psp_hbm.at[idxSf.at[sl]], B[2], sem)
        pltpu.async_copy(psc_hbm.at[idxSf.at[sl]], B[3], sem)
        pltpu.async_copy(pap_hbm.at[idxAf.at[sl]], B[4], sem)
        pltpu.async_copy(pac_hbm.at[idxAf.at[sl]], B[5], sem)

    def wait_gathers(b):
        dsl = pl.ds(0, CB_B)
        for t in range(6):
            pltpu.make_async_copy(pdp_hbm.at[idxDf.at[dsl]],
                                  bufs[b][t], semsG[b]).wait()

    def issue_writes(j, b):
        st = base + j * CB_B
        pltpu.async_copy(bufs[b][0], h1p_hbm.at[pl.ds(st, CB_B)], semsW[b])
        pltpu.async_copy(bufs[b][1], h1c_hbm.at[pl.ds(st, CB_B)], semsW[b])

    def wait_writes(b):
        dsl = pl.ds(0, CB_B)
        pltpu.make_async_copy(bufs[b][0], h1p_hbm.at[dsl], semsW[b]).wait()
        pltpu.make_async_copy(bufs[b][1], h1c_hbm.at[dsl], semsW[b]).wait()

    def compute(b, acc):
        B = bufs[b]

        def row_body(j, a):
            a = list(a)
            for k in range(8):
                sl = pl.ds(k * 16, 16)
                v = B[0][j, sl] + B[2][j, sl] + B[4][j, sl]
                B[0][j, sl] = v
                a[k] = a[k] + v
                a[8 + k] = a[8 + k] + v * v
                w = B[1][j, sl] + B[3][j, sl] + B[5][j, sl]
                B[1][j, sl] = w
                a[16 + k] = a[16 + k] + w
                a[24 + k] = a[24 + k] + w * w
            return tuple(a)

        return lax.fori_loop(0, CB_B, row_body, acc)

    zero = jnp.zeros((16,), jnp.float32)
    init = tuple(zero for _ in range(32))

    issue_gathers(0, 0)
    issue_gathers(1, 1)

    def step(t, acc):
        for b in (0, 1):
            j = 2 * t + b
            wait_gathers(b)
            acc = compute(b, acc)
            issue_writes(j, b)

            @pl.when(j + 2 < NCH_B)
            def _():
                wait_writes(b)
                issue_gathers(j + 2, b)
        return acc

    acc = lax.fori_loop(0, NCH_B // 2, step, init)
    wait_writes(0)
    wait_writes(1)
    for k in range(8):
        sl = pl.ds(k * 16, 16)
        stage[0, sl] = acc[k]
        stage[1, sl] = acc[8 + k]
        stage[2, sl] = acc[16 + k]
        stage[3, sl] = acc[24 + k]
        for r in range(4, 8):
            stage[r, sl] = zero
    pltpu.sync_copy(stage, stats_hbm.at[wid])


_gather_sum_call = pl.kernel(
    _sc_gather_sum,
    out_type=[
        jax.ShapeDtypeStruct((E, D), jnp.float32),
        jax.ShapeDtypeStruct((E, D), jnp.float32),
        jax.ShapeDtypeStruct((NW, 8, D), jnp.float32),
    ],
    mesh=_mesh,
    scratch_types=(
        [pltpu.VMEM((EPW,), jnp.int32)] * 3
        + [pltpu.VMEM((CB_B, D), jnp.float32)] * 12
        + [pltpu.VMEM((8, D), jnp.float32)]
        + [pltpu.SemaphoreType.DMA] * 4
    ),
    name="sc_gather_sum",
)


def _sc_scatter(h2p_hbm, h2c_hbm, coef_hbm, dst_hbm, src_hbm,
                aggp_hbm, aggc_hbm,
                buf0, buf1, idx0, idx1, coefv, acc,
                semR0, semR1, semS0, semS1):
    c = lax.axis_index("c")
    s = lax.axis_index("s")
    bufs = [buf0, buf1]
    idxs = [idx0, idx1]
    semR = [semR0, semR1]
    semS = [semS0, semS1]

    def zero_buf(j, _):
        for k in range(8):
            buf0[j, pl.ds(k * 16, 16)] = jnp.zeros((16,), jnp.float32)
        return 0
    lax.fori_loop(0, CB_D, zero_buf, 0)

    for i in range(8):
        ch = s + i * NS

        @pl.when(ch < NCH_N)
        def _():
            pltpu.sync_copy(buf0, acc.at[pl.ds(ch * CB_N, CB_N)])

    plsc.subcore_barrier()

    pltpu.sync_copy(coef_hbm, coefv)

    def scatter_loop(h2_hbm, idx_hbm, arow, crow):
        av = [coefv[arow, pl.ds(k * 16, 16)] for k in range(8)]
        cv = [coefv[crow, pl.ds(k * 16, 16)] for k in range(8)]
        ebase = s * EPT

        def issue_read(j, b):
            st = ebase + j * CB_D
            pltpu.async_copy(idx_hbm.at[pl.ds(st, CB_D)], idxs[b], semR[b])
            pltpu.async_copy(h2_hbm.at[pl.ds(st, CB_D)], bufs[b], semR[b])

        def wait_read(b):
            dsl = pl.ds(0, CB_D)
            pltpu.make_async_copy(idx_hbm.at[dsl], idxs[b], semR[b]).wait()
            pltpu.make_async_copy(h2_hbm.at[dsl], bufs[b], semR[b]).wait()

        def wait_scat(b):
            pltpu.make_async_copy(h2_hbm.at[pl.ds(0, CB_D)], bufs[b],
                                  semS[b]).wait()

        def compute(b):
            def row(j, _):
                for k in range(8):
                    sl = pl.ds(k * 16, 16)
                    bufs[b][j, sl] = jnp.maximum(
                        bufs[b][j, sl] * av[k] + cv[k], 0.0)
                return 0
            lax.fori_loop(0, CB_D, row, 0)

        issue_read(0, 0)
        issue_read(1, 1)

        def step(t, _):
            for b in (0, 1):
                j = 2 * t + b
                wait_read(b)
                compute(b)
                pltpu.async_copy(bufs[b], acc.at[idxs[b]], semS[b], add=True)

                @pl.when(j + 2 < NCH_D)
                def _():
                    wait_scat(b)
                    issue_read(j + 2, b)
            return 0

        lax.fori_loop(0, NCH_D // 2, step, 0)
        wait_scat(0)
        wait_scat(1)

    @pl.when(c == 0)
    def _():
        scatter_loop(h2p_hbm, dst_hbm, 0, 1)

    @pl.when(c == 1)
    def _():
        scatter_loop(h2c_hbm, src_hbm, 2, 3)

    plsc.subcore_barrier()

    def agg_out(out_hbm):
        for i in range(8):
            ch = s + i * NS

            @pl.when(ch < NCH_N)
            def _():
                pltpu.sync_copy(acc.at[pl.ds(ch * CB_N, CB_N)], buf0)
                pltpu.sync_copy(buf0, out_hbm.at[pl.ds(ch * CB_N, CB_N)])

    @pl.when(c == 0)
    def _():
        agg_out(aggp_hbm)

    @pl.when(c == 1)
    def _():
        agg_out(aggc_hbm)


_scatter_call = pl.kernel(
    _sc_scatter,
    out_type=[
        jax.ShapeDtypeStruct((N, D), jnp.float32),
        jax.ShapeDtypeStruct((N, D), jnp.float32),
    ],
    mesh=_mesh,
    scratch_types=[
        pltpu.VMEM((CB_D, D), jnp.float32),
        pltpu.VMEM((CB_D, D), jnp.float32),
        pltpu.VMEM((CB_D,), jnp.int32),
        pltpu.VMEM((CB_D,), jnp.int32),
        pltpu.VMEM((4, D), jnp.float32),
        pltpu.VMEM_SHARED((N, D), jnp.float32),
        pltpu.SemaphoreType.DMA,
        pltpu.SemaphoreType.DMA,
        pltpu.SemaphoreType.DMA,
        pltpu.SemaphoreType.DMA,
    ],
    name="sc_scatter",
)


def _tc_proj(nodes, wdp, wdc, wsp, wsc, et, wap, wac, pdp, pdc, psp, psc,
             pap, pac):
    n = nodes[...]
    pdp[...] = jnp.dot(n, wdp[...], preferred_element_type=jnp.float32)
    pdc[...] = jnp.dot(n, wdc[...], preferred_element_type=jnp.float32)
    psp[...] = jnp.dot(n, wsp[...], preferred_element_type=jnp.float32)
    psc[...] = jnp.dot(n, wsc[...], preferred_element_type=jnp.float32)
    e = et[...]
    pap[...] = jnp.dot(e, wap[...], preferred_element_type=jnp.float32)
    pac[...] = jnp.dot(e, wac[...], preferred_element_type=jnp.float32)


def _run_tc_proj(nodes, wdp, wdc, wsp, wsc, et, wap, wac):
    return pl.pallas_call(
        _tc_proj,
        out_shape=[
            jax.ShapeDtypeStruct((N, D), jnp.float32),
            jax.ShapeDtypeStruct((N, D), jnp.float32),
            jax.ShapeDtypeStruct((N, D), jnp.float32),
            jax.ShapeDtypeStruct((N, D), jnp.float32),
            jax.ShapeDtypeStruct((MAX_EDGES, D), jnp.float32),
            jax.ShapeDtypeStruct((MAX_EDGES, D), jnp.float32),
        ],
        name="tc_proj",
    )(nodes, wdp, wdc, wsp, wsc, et, wap, wac)


def _tc_mlp2(h1p, h1c, stats1, w2p, b2p, g2p, be2p, g1p, be1p,
             w2c, b2c, g2c, be2c, g1c, be1c,
             h2p, h2c, stats2, coef):
    i = pl.program_id(0)
    s1 = jnp.sum(stats1[...], axis=0)
    fE = float(E)

    def bn_coef(su, sq, g, be):
        mu = su / fE
        var = sq / fE - mu * mu
        a = g * lax.rsqrt(var + EPS)
        return a, be - a * mu

    a1p, c1p = bn_coef(s1[0:1, :], s1[1:2, :], g1p[...], be1p[...])
    a1c, c1c = bn_coef(s1[2:3, :], s1[3:4, :], g1c[...], be1c[...])

    zp = jnp.maximum(a1p * h1p[...] + c1p, 0.0)
    zc = jnp.maximum(a1c * h1c[...] + c1c, 0.0)
    vp = jnp.dot(zp, w2p[...], preferred_element_type=jnp.float32) + b2p[...]
    vc = jnp.dot(zc, w2c[...], preferred_element_type=jnp.float32) + b2c[...]
    h2p[...] = vp
    h2c[...] = vc

    bs = jnp.concatenate(
        [jnp.concatenate([jnp.sum(vp, 0, keepdims=True),
                          jnp.sum(vc, 0, keepdims=True)], axis=1),
         jnp.concatenate([jnp.sum(vp * vp, 0, keepdims=True),
                          jnp.sum(vc * vc, 0, keepdims=True)], axis=1)],
        axis=0)

    @pl.when(i == 0)
    def _():
        stats2[...] = bs

    @pl.when(i > 0)
    def _():
        stats2[...] += bs

    tot = stats2[...]
    a2p, c2p = bn_coef(tot[0:1, :D], tot[1:2, :D], g2p[...], be2p[...])
    a2c, c2c = bn_coef(tot[0:1, D:], tot[1:2, D:], g2c[...], be2c[...])
    coef[...] = jnp.concatenate([a2p, c2p, a2c, c2c], axis=0)


def _run_tc_mlp2(h1p, h1c, stats1, w2p, b2p, g2p, be2p, g1p, be1p,
                 w2c, b2c, g2c, be2c, g1c, be1c):
    blk = pl.BlockSpec((BE, D), lambda i: (i, 0))
    whole2 = pl.BlockSpec((1, D), lambda i: (0, 0))
    wmat = pl.BlockSpec((D, D), lambda i: (0, 0))
    return pl.pallas_call(
        _tc_mlp2,
        grid=(NBLK,),
        in_specs=[blk, blk, pl.BlockSpec((NW, 8, D), lambda i: (0, 0, 0)),
                  wmat, whole2, whole2, whole2, whole2, whole2,
                  wmat, whole2, whole2, whole2, whole2, whole2],
        out_specs=[blk, blk,
                   pl.BlockSpec((2, 2 * D), lambda i: (0, 0)),
                   pl.BlockSpec((4, D), lambda i: (0, 0))],
        out_shape=[
            jax.ShapeDtypeStruct((E, D), jnp.float32),
            jax.ShapeDtypeStruct((E, D), jnp.float32),
            jax.ShapeDtypeStruct((2, 2 * D), jnp.float32),
            jax.ShapeDtypeStruct((4, D), jnp.float32),
        ],
        name="tc_mlp2",
    )(h1p, h1c, stats1, w2p, b2p, g2p, be2p, g1p, be1p,
      w2c, b2c, g2c, be2c, g1c, be1c)


BN = 1000
NBN = N // BN


def _tc_update1(nodes, aggp, aggc, degp, degc, wf, bf, h_out, stats):
    i = pl.program_id(0)
    sump = jnp.sum(degp[...], axis=1, keepdims=True)
    sumc = jnp.sum(degc[...], axis=1, keepdims=True)
    invp = jnp.where(sump > 0, float(D) / sump, 0.0)
    invc = jnp.where(sumc > 0, float(D) / sumc, 0.0)
    s = nodes[...] + aggp[...] * invp + aggc[...] * invc
    h = jnp.dot(s, wf[...], preferred_element_type=jnp.float32) + bf[...]
    h_out[...] = h
    bs = jnp.concatenate([jnp.sum(h, 0, keepdims=True),
                          jnp.sum(h * h, 0, keepdims=True)], axis=0)

    @pl.when(i == 0)
    def _():
        stats[...] = bs

    @pl.when(i > 0)
    def _():
        stats[...] += bs


def _tc_update2(nodes, h, stats, gf, bef, nodes_new):
    fN = float(N)
    tot = stats[...]
    mu = tot[0:1, :] / fN
    var = tot[1:2, :] / fN - mu * mu
    a = gf[...] * lax.rsqrt(var + EPS)
    c = bef[...] - a * mu
    upd = jnp.maximum(a * h[...] + c, 0.0)
    nodes_new[...] = nodes[...] + upd


def _run_tc_update(nodes, aggp, aggc, degp, degc, wf, bf, gf, bef):
    blk = pl.BlockSpec((BN, D), lambda i: (i, 0))
    whole2 = pl.BlockSpec((1, D), lambda i: (0, 0))
    h, stats = pl.pallas_call(
        _tc_update1,
        grid=(NBN,),
        in_specs=[blk, blk, blk, blk, blk,
                  pl.BlockSpec((D, D), lambda i: (0, 0)), whole2],
        out_specs=[blk, pl.BlockSpec((2, D), lambda i: (0, 0))],
        out_shape=[
            jax.ShapeDtypeStruct((N, D), jnp.float32),
            jax.ShapeDtypeStruct((2, D), jnp.float32),
        ],
        name="tc_update1",
    )(nodes, aggp, aggc, degp, degc, wf, bf)
    return pl.pallas_call(
        _tc_update2,
        grid=(NBN,),
        in_specs=[blk, blk, pl.BlockSpec((2, D), lambda i: (0, 0)),
                  whole2, whole2],
        out_specs=blk,
        out_shape=jax.ShapeDtypeStruct((N, D), jnp.float32),
        name="tc_update2",
    )(nodes, h, stats, gf, bef)


def _tc_segmax(nodes, batch2d, gmax):
    g = pl.program_id(0)
    neg = jnp.float32(-jnp.inf)
    m = jnp.where(batch2d[...] == g, nodes[...], neg)
    gmax[...] = jnp.max(m, axis=0, keepdims=True)[None]


def _run_tc_segmax(nodes, batch2d):
    out = pl.pallas_call(
        _tc_segmax,
        grid=(NUM_GRAPHS,),
        in_specs=[pl.BlockSpec((N, D), lambda g: (0, 0)),
                  pl.BlockSpec((N, 1), lambda g: (0, 0))],
        out_specs=pl.BlockSpec((1, 1, D), lambda g: (g, 0, 0)),
        out_shape=jax.ShapeDtypeStruct((NUM_GRAPHS, 1, D), jnp.float32),
        name="tc_segmax",
    )(nodes, batch2d)
    return out.reshape(NUM_GRAPHS, D)


def kernel(x, edge_index, edge_attr, batch, node_table, edge_table,
           W1_p, b1_p, g1_p, be1_p, W2_p, b2_p, g2_p, be2_p,
           W1_c, b1_c, g1_c, be1_c, W2_c, b2_c, g2_c, be2_c,
           Wf, bf, gf, bef):
    dst = edge_index[1]
    src = edge_index[0]
    attr = edge_attr.astype(jnp.int32)
    xi = x.astype(jnp.int32)
    batch2d = batch.astype(jnp.int32)[:, None]

    wdp = W1_p[0:D, :]
    wdc = W1_c[D:2 * D, :]
    wsp = W1_p[D:2 * D, :]
    wsc = W1_c[0:D, :]
    wap = W1_p[2 * D:, :]
    wac = W1_c[2 * D:, :]

    r2 = lambda v: v[None, :]

    nodes, degp, degc = _prologue_call(xi, node_table, dst, src)

    gmax = None
    for _ in range(NUM_ITER):
        pdp, pdc, psp, psc, pap, pac = _run_tc_proj(
            nodes, wdp, wdc, wsp, wsc, edge_table, wap, wac)
        pap = pap + r2(b1_p)
        pac = pac + r2(b1_c)
        h1p, h1c, stats1 = _gather_sum_call(
            pdp, pdc, psp, psc, pap, pac, dst, src, attr)
        h2p, h2c, _, coef = _run_tc_mlp2(
            h1p, h1c, stats1,
            W2_p, r2(b2_p), r2(g2_p), r2(be2_p), r2(g1_p), r2(be1_p),
            W2_c, r2(b2_c), r2(g2_c), r2(be2_c), r2(g1_c), r2(be1_c))
        aggp, aggc = _scatter_call(h2p, h2c, coef, dst, src)
        nodes = _run_tc_update(nodes, aggp, aggc, degp, degc,
                               Wf, r2(bf), r2(gf), r2(bef))
    return _run_tc_segmax(nodes, batch2d)
